# baseline (speedup 1.0000x reference)
import jax
import jax.numpy as jnp
from jax import lax
from jax.experimental import pallas as pl
from jax.experimental.pallas import tpu as pltpu
from jax.experimental.pallas import tpu_sc as plsc

D = 128
N_HDM = 2016
N_BAR = 288
N_ROWS = N_HDM + N_BAR

NC, NS, L = 2, 16, 16
NW = NC * NS
NT = 4096 * 200
PER_W = NT // NW
G = 1024
N_CHUNK = PER_W // G
SG = 128
NJ = G // SG


def _prep_body(hw, dw, mw, bw, pw, pb, out):
    f32 = jnp.float32
    hi = lax.Precision.HIGHEST
    p = pw[:]
    th = jnp.dot(hw[:], p[:, 0:32].T, precision=hi, preferred_element_type=f32)
    td = jnp.dot(dw[:], p[:, 32:64].T, precision=hi, preferred_element_type=f32)
    tm = jnp.dot(mw[:], p[:, 64:96].T, precision=hi, preferred_element_type=f32)
    tb = jnp.dot(bw[:], p[:, 96:128].T, precision=hi, preferred_element_type=f32)
    r24 = lax.broadcasted_iota(jnp.int32, (N_HDM, 24), 0)
    c24 = lax.broadcasted_iota(jnp.int32, (N_HDM, 24), 1)
    oh_h = jnp.where(r24 % 24 == c24, 1.0, 0.0).astype(f32)
    r7 = lax.broadcasted_iota(jnp.int32, (N_HDM, 7), 0)
    c7 = lax.broadcasted_iota(jnp.int32, (N_HDM, 7), 1)
    oh_d = jnp.where((r7 // 24) % 7 == c7, 1.0, 0.0).astype(f32)
    r12 = lax.broadcasted_iota(jnp.int32, (N_HDM, 12), 0)
    c12 = lax.broadcasted_iota(jnp.int32, (N_HDM, 12), 1)
    oh_m = jnp.where(r12 // 168 == c12, 1.0, 0.0).astype(f32)
    hdm = (jnp.dot(oh_h, th, precision=hi, preferred_element_type=f32)
           + jnp.dot(oh_d, td, precision=hi, preferred_element_type=f32)
           + jnp.dot(oh_m, tm, precision=hi, preferred_element_type=f32)
           + pb[:])

    i32 = jnp.int32
    r128 = lax.broadcasted_iota(i32, (D, DP), 0)
    c64 = lax.broadcasted_iota(i32, (D, DP), 1)
    n0 = 32 * (c64 // 16) + (c64 % 16)
    s0 = (r128 == n0).astype(f32)
    s1 = (r128 == n0 + 16).astype(f32)

    def pack(block):
        a = lax.bitcast_convert_type(
            jnp.dot(block, s0, precision=hi, preferred_element_type=f32), i32)
        b = lax.bitcast_convert_type(
            jnp.dot(block, s1, precision=hi, preferred_element_type=f32), i32)
        lo = lax.shift_right_logical(
            a + 0x7FFF + (lax.shift_right_logical(a, 16) & 1), 16)
        hi_bits = (b + 0x7FFF + (lax.shift_right_logical(b, 16) & 1)) & jnp.int32(-65536)
        return lo | hi_bits

    out[0:N_HDM, :] = pack(hdm)
    out[N_HDM:N_ROWS, :] = pack(tb)


NBUF = 2
DP = D // 2


def _sc_body(tab, hour, day, month, bar, out,
             tabsp, hbuf, dbuf, mbuf, bbuf, cidx, bidx,
             rc0, rc1, rb0, rb1, fout0, fout1,
             gsem0, gsem1, ssem0, ssem1):
    c = lax.axis_index("c")
    s = lax.axis_index("s")
    wid = s * NC + c
    base_w = wid * PER_W
    rc = [rc0, rc1]
    rb = [rb0, rb1]
    fout = [fout0, fout1]
    gsem = [gsem0, gsem1]
    ssem = [ssem0, ssem1]
    f32 = jnp.float32
    himask = jnp.int32(-65536)

    @pl.when(s == 0)
    def _():
        pltpu.sync_copy(tab, tabsp)

    plsc.subcore_barrier()

    def chunk(g, carry):
        base = base_w + g * G
        pltpu.sync_copy(hour.at[pl.ds(base, G)], hbuf)
        pltpu.sync_copy(day.at[pl.ds(base, G)], dbuf)
        pltpu.sync_copy(month.at[pl.ds(base, G)], mbuf)
        pltpu.sync_copy(bar.at[pl.ds(base, G)], bbuf)
        for i in range(G // L):
            sl = pl.ds(i * L, L)
            j, o = i // (SG // L), (i % (SG // L)) * L
            osl = pl.ds(o, L)
            cidx[j, osl] = hbuf[sl] + dbuf[sl] * 24 + mbuf[sl] * 168
            bidx[j, osl] = jnp.minimum(bbuf[sl], N_BAR - 1) + N_HDM

        def gather_pair(j):
            slot = j % NBUF
            hc = pltpu.async_copy(tabsp.at[cidx.at[j]], rc[slot], gsem[slot])
            hb = pltpu.async_copy(tabsp.at[bidx.at[j]], rb[slot], gsem[slot])
            return hc, hb

        def drain_store(fb):
            pltpu.make_async_copy(fout[fb], out.at[pl.ds(base, SG)],
                                  ssem[fb]).wait()

        hs = [None] * NJ
        hs[0] = gather_pair(0)
        for j in range(NJ):
            slot = j % NBUF
            fb = j % 2
            if j + 1 < NJ:
                hs[j + 1] = gather_pair(j + 1)
            hs[j][0].wait()
            hs[j][1].wait()
            if j < 2:
                @pl.when(g > 0)
                def _():
                    drain_store(fb)
            else:
                drain_store(fb)

            @plsc.parallel_loop(0, SG, 1, unroll=2)
            def _(t):
                for k in range(DP // L):
                    ksl = pl.ds(k * L, L)
                    vc = rc[slot][t, ksl]
                    vb = rb[slot][t, ksl]
                    lo = (lax.bitcast_convert_type(vc << 16, f32)
                          + lax.bitcast_convert_type(vb << 16, f32))
                    hi = (lax.bitcast_convert_type(vc & himask, f32)
                          + lax.bitcast_convert_type(vb & himask, f32))
                    fout[fb][t, pl.ds(2 * k * L, L)] = lo
                    fout[fb][t, pl.ds((2 * k + 1) * L, L)] = hi

            pltpu.async_copy(fout[fb], out.at[pl.ds(base + j * SG, SG)],
                             ssem[fb])
        return carry

    lax.fori_loop(0, N_CHUNK, chunk, 0)
    for fb in range(2):
        pltpu.make_async_copy(fout[fb], out.at[pl.ds(0, SG)],
                              ssem[fb]).wait()


def _build_sc():
    return pl.kernel(
        _sc_body,
        out_type=jax.ShapeDtypeStruct((NT, D), jnp.float32),
        mesh=plsc.VectorSubcoreMesh(core_axis_name="c", subcore_axis_name="s",
                                    num_cores=NC, num_subcores=NS),
        compiler_params=pltpu.CompilerParams(use_tc_tiling_on_sc=False),
        scratch_types=(
            [pltpu.VMEM_SHARED((N_ROWS, DP), jnp.int32)]
            + [pltpu.VMEM((G,), jnp.int32)] * 4
            + [pltpu.VMEM((NJ, SG), jnp.int32)] * 2
            + [pltpu.VMEM((SG, DP), jnp.int32)] * 4
            + [pltpu.VMEM((SG, D), jnp.float32)] * 2
            + [pltpu.SemaphoreType.DMA] * 4
        ),
    )


def kernel(hour, day, month, bar_idx, hour_W, day_W, month_W, bar_W, proj_W, proj_b):
    f32 = jnp.float32
    tpack = pl.pallas_call(
        _prep_body,
        out_shape=jax.ShapeDtypeStruct((N_ROWS, DP), jnp.int32),
    )(hour_W.astype(f32), day_W.astype(f32), month_W.astype(f32),
      bar_W.astype(f32), proj_W.astype(f32), proj_b.astype(f32).reshape(1, D))
    i32 = jnp.int32
    h = hour.reshape(-1).astype(i32)
    d = day.reshape(-1).astype(i32)
    m = month.reshape(-1).astype(i32)
    b = bar_idx.reshape(-1).astype(i32)
    out = _build_sc()(tpack, h, d, m, b)
    return out.reshape(hour.shape[0], hour.shape[1], D)


# speedup vs baseline: 1.1860x; 1.1860x over previous
import jax
import jax.numpy as jnp
from jax import lax
from jax.experimental import pallas as pl
from jax.experimental.pallas import tpu as pltpu
from jax.experimental.pallas import tpu_sc as plsc

D = 128
N_HDM = 2016
N_BAR = 288
N_ROWS = N_HDM + N_BAR

NC, NS, L = 2, 16, 16
NW = NC * NS
NT = 4096 * 200
PER_W = NT // NW
G = 1280
N_CHUNK = PER_W // G
SG = 128
NJ = G // SG


def _prep_body(hw, dw, mw, bw, pw, pb, out):
    f32 = jnp.float32
    hi = lax.Precision.HIGHEST
    p = pw[:]
    th = jnp.dot(hw[:], p[:, 0:32].T, precision=hi, preferred_element_type=f32)
    td = jnp.dot(dw[:], p[:, 32:64].T, precision=hi, preferred_element_type=f32)
    tm = jnp.dot(mw[:], p[:, 64:96].T, precision=hi, preferred_element_type=f32)
    tb = jnp.dot(bw[:], p[:, 96:128].T, precision=hi, preferred_element_type=f32)
    r24 = lax.broadcasted_iota(jnp.int32, (N_HDM, 24), 0)
    c24 = lax.broadcasted_iota(jnp.int32, (N_HDM, 24), 1)
    oh_h = jnp.where(r24 % 24 == c24, 1.0, 0.0).astype(f32)
    r7 = lax.broadcasted_iota(jnp.int32, (N_HDM, 7), 0)
    c7 = lax.broadcasted_iota(jnp.int32, (N_HDM, 7), 1)
    oh_d = jnp.where((r7 // 24) % 7 == c7, 1.0, 0.0).astype(f32)
    r12 = lax.broadcasted_iota(jnp.int32, (N_HDM, 12), 0)
    c12 = lax.broadcasted_iota(jnp.int32, (N_HDM, 12), 1)
    oh_m = jnp.where(r12 // 168 == c12, 1.0, 0.0).astype(f32)
    hdm = (jnp.dot(oh_h, th, precision=hi, preferred_element_type=f32)
           + jnp.dot(oh_d, td, precision=hi, preferred_element_type=f32)
           + jnp.dot(oh_m, tm, precision=hi, preferred_element_type=f32)
           + pb[:])

    i32 = jnp.int32
    r128 = lax.broadcasted_iota(i32, (D, DP), 0)
    c64 = lax.broadcasted_iota(i32, (D, DP), 1)
    n0 = 32 * (c64 // 16) + (c64 % 16)
    s0 = (r128 == n0).astype(f32)
    s1 = (r128 == n0 + 16).astype(f32)

    def pack(block):
        a = lax.bitcast_convert_type(
            jnp.dot(block, s0, precision=hi, preferred_element_type=f32), i32)
        b = lax.bitcast_convert_type(
            jnp.dot(block, s1, precision=hi, preferred_element_type=f32), i32)
        lo = lax.shift_right_logical(
            a + 0x7FFF + (lax.shift_right_logical(a, 16) & 1), 16)
        hi_bits = (b + 0x7FFF + (lax.shift_right_logical(b, 16) & 1)) & jnp.int32(-65536)
        return lo | hi_bits

    out[0:N_HDM, :] = pack(hdm)
    out[N_HDM:N_ROWS, :] = pack(tb)


NBUF = 2
DP = D // 2


def _sc_body(tab, hour, day, month, bar, out,
             tabsp, h0, d0, m0, b0, h1, d1, m1, b1, cidx, bidx,
             rc0, rc1, rb0, rb1, fout0, fout1,
             gsem0, gsem1, ssem0, ssem1, isem0, isem1):
    c = lax.axis_index("c")
    s = lax.axis_index("s")
    wid = s * NC + c
    base_w = wid * PER_W
    idxsets = [(h0, d0, m0, b0, isem0), (h1, d1, m1, b1, isem1)]
    rc = [rc0, rc1]
    rb = [rb0, rb1]
    fout = [fout0, fout1]
    gsem = [gsem0, gsem1]
    ssem = [ssem0, ssem1]
    f32 = jnp.float32
    himask = jnp.int32(-65536)

    @pl.when(s == 0)
    def _():
        pltpu.sync_copy(tab, tabsp)

    plsc.subcore_barrier()

    def issue_idx(base, setk):
        hb, db, mb, bb, isem = idxsets[setk]
        pltpu.async_copy(hour.at[pl.ds(base, G)], hb, isem)
        pltpu.async_copy(day.at[pl.ds(base, G)], db, isem)
        pltpu.async_copy(month.at[pl.ds(base, G)], mb, isem)
        pltpu.async_copy(bar.at[pl.ds(base, G)], bb, isem)

    def drain_idx(setk):
        hb, db, mb, bb, isem = idxsets[setk]
        for buf in (hb, db, mb, bb):
            pltpu.make_async_copy(hour.at[pl.ds(0, G)], buf, isem).wait()

    def compute_indices(setk):
        hb, db, mb, bb, _ = idxsets[setk]
        for i in range(G // L):
            sl = pl.ds(i * L, L)
            j, o = i // (SG // L), (i % (SG // L)) * L
            osl = pl.ds(o, L)
            cidx[j, osl] = hb[sl] + db[sl] * 24 + mb[sl] * 168
            bidx[j, osl] = jnp.minimum(bb[sl], N_BAR - 1) + N_HDM

    def pipeline(base, guard):
        def gather_pair(j):
            slot = j % NBUF
            hc = pltpu.async_copy(tabsp.at[cidx.at[j]], rc[slot], gsem[slot])
            hb = pltpu.async_copy(tabsp.at[bidx.at[j]], rb[slot], gsem[slot])
            return hc, hb

        def drain_store(fb):
            pltpu.make_async_copy(fout[fb], out.at[pl.ds(base, SG)],
                                  ssem[fb]).wait()

        hs = [None] * NJ
        hs[0] = gather_pair(0)
        for j in range(NJ):
            slot = j % NBUF
            fb = j % 2
            if j + 1 < NJ:
                hs[j + 1] = gather_pair(j + 1)
            hs[j][0].wait()
            hs[j][1].wait()
            if j < 2 and guard is not None:
                @pl.when(guard)
                def _():
                    drain_store(fb)
            else:
                drain_store(fb)

            @plsc.parallel_loop(0, SG, 1, unroll=2)
            def _(t):
                for k in range(DP // L):
                    ksl = pl.ds(k * L, L)
                    vc = rc[slot][t, ksl]
                    vb = rb[slot][t, ksl]
                    lo = (lax.bitcast_convert_type(vc << 16, f32)
                          + lax.bitcast_convert_type(vb << 16, f32))
                    hi = (lax.bitcast_convert_type(vc & himask, f32)
                          + lax.bitcast_convert_type(vb & himask, f32))
                    fout[fb][t, pl.ds(2 * k * L, L)] = lo
                    fout[fb][t, pl.ds((2 * k + 1) * L, L)] = hi

            pltpu.async_copy(fout[fb], out.at[pl.ds(base + j * SG, SG)],
                             ssem[fb])

    issue_idx(base_w, 0)

    def dchunk(gg, carry):
        base0 = base_w + (2 * gg) * G
        base1 = base0 + G
        drain_idx(0)
        compute_indices(0)
        issue_idx(base1, 1)
        pipeline(base0, guard=(gg > 0))
        drain_idx(1)
        compute_indices(1)

        @pl.when(gg < N_CHUNK // 2 - 1)
        def _():
            issue_idx(base1 + G, 0)

        pipeline(base1, guard=None)
        return carry

    lax.fori_loop(0, N_CHUNK // 2, dchunk, 0)
    for fb in range(2):
        pltpu.make_async_copy(fout[fb], out.at[pl.ds(0, SG)],
                              ssem[fb]).wait()


def _build_sc():
    return pl.kernel(
        _sc_body,
        out_type=jax.ShapeDtypeStruct((NT, D), jnp.float32),
        mesh=plsc.VectorSubcoreMesh(core_axis_name="c", subcore_axis_name="s",
                                    num_cores=NC, num_subcores=NS),
        compiler_params=pltpu.CompilerParams(use_tc_tiling_on_sc=False),
        scratch_types=(
            [pltpu.VMEM_SHARED((N_ROWS, DP), jnp.int32)]
            + [pltpu.VMEM((G,), jnp.int32)] * 8
            + [pltpu.VMEM((NJ, SG), jnp.int32)] * 2
            + [pltpu.VMEM((SG, DP), jnp.int32)] * 4
            + [pltpu.VMEM((SG, D), jnp.float32)] * 2
            + [pltpu.SemaphoreType.DMA] * 6
        ),
    )


def kernel(hour, day, month, bar_idx, hour_W, day_W, month_W, bar_W, proj_W, proj_b):
    f32 = jnp.float32
    tpack = pl.pallas_call(
        _prep_body,
        out_shape=jax.ShapeDtypeStruct((N_ROWS, DP), jnp.int32),
    )(hour_W.astype(f32), day_W.astype(f32), month_W.astype(f32),
      bar_W.astype(f32), proj_W.astype(f32), proj_b.astype(f32).reshape(1, D))
    i32 = jnp.int32
    h = hour.reshape(-1).astype(i32)
    d = day.reshape(-1).astype(i32)
    m = month.reshape(-1).astype(i32)
    b = bar_idx.reshape(-1).astype(i32)
    out = _build_sc()(tpack, h, d, m, b)
    return out.reshape(hour.shape[0], hour.shape[1], D)
